# baseline (speedup 1.0000x reference)
import functools

import jax
import jax.numpy as jnp
from jax import lax
from jax.experimental import pallas as pl
from jax.experimental.pallas import tpu as pltpu
from jax.experimental.pallas import tpu_sc as plsc

MAX_POS = 4096
HIDDEN = 1024
EPS = 1e-12

NC = 2
NS = 16
NW = NC * NS

CHUNK = 32
NBUF = 3


def _sc_gather(pos_table, ids_flat):
    n_tokens = ids_flat.shape[0]
    b_per_w = n_tokens // NW
    n_ch = b_per_w // CHUNK
    mesh = plsc.VectorSubcoreMesh(core_axis_name="c", subcore_axis_name="s")

    @functools.partial(
        pl.kernel,
        mesh=mesh,
        out_type=jax.ShapeDtypeStruct((n_tokens, HIDDEN), jnp.float32),
        scratch_types=(
            [pltpu.VMEM((b_per_w,), jnp.int32)]
            + [pltpu.VMEM((CHUNK, HIDDEN), jnp.float32)] * NBUF
            + [pltpu.SemaphoreType.DMA] * (2 * NBUF)
        ),
    )
    def k(table_hbm, idx_hbm, out_hbm, idx_v, *scratch):
        bufs = list(scratch[:NBUF])
        gsem = list(scratch[NBUF : 2 * NBUF])
        ssem = list(scratch[2 * NBUF : 3 * NBUF])
        wid = lax.axis_index("s") * NC + lax.axis_index("c")
        base = wid * b_per_w
        pltpu.sync_copy(idx_hbm.at[pl.ds(base, b_per_w)], idx_v)

        gathers = [None] * n_ch
        stores = [None] * n_ch
        for c in range(n_ch):
            p = c % NBUF
            if c >= NBUF:
                stores[c - NBUF].wait()
            gathers[c] = pltpu.async_copy(
                table_hbm.at[idx_v.at[pl.ds(c * CHUNK, CHUNK)]],
                bufs[p],
                gsem[p],
            )
            if c >= 1:
                q = (c - 1) % NBUF
                gathers[c - 1].wait()
                stores[c - 1] = pltpu.async_copy(
                    bufs[q],
                    out_hbm.at[pl.ds(base + (c - 1) * CHUNK, CHUNK)],
                    ssem[q],
                )
        gathers[n_ch - 1].wait()
        stores[n_ch - 1] = pltpu.async_copy(
            bufs[(n_ch - 1) % NBUF],
            out_hbm.at[pl.ds(base + (n_ch - 1) * CHUNK, CHUNK)],
            ssem[(n_ch - 1) % NBUF],
        )
        for c in range(max(0, n_ch - NBUF), n_ch):
            stores[c].wait()

    return k(pos_table, ids_flat)


def _tc_add_ln(x, pe, gamma, beta):
    n = x.shape[0]
    bt = 512
    grid = (n // bt,)

    def body(x_ref, p_ref, g_ref, b_ref, o_ref):
        e = x_ref[...] + p_ref[...]
        m = jnp.mean(e, axis=1, keepdims=True)
        d = e - m
        v = jnp.mean(d * d, axis=1, keepdims=True)
        o_ref[...] = d * lax.rsqrt(v + EPS) * g_ref[...] + b_ref[...]

    return pl.pallas_call(
        body,
        grid=grid,
        in_specs=[
            pl.BlockSpec((bt, HIDDEN), lambda i: (i, 0)),
            pl.BlockSpec((bt, HIDDEN), lambda i: (i, 0)),
            pl.BlockSpec((1, HIDDEN), lambda i: (0, 0)),
            pl.BlockSpec((1, HIDDEN), lambda i: (0, 0)),
        ],
        out_specs=pl.BlockSpec((bt, HIDDEN), lambda i: (i, 0)),
        out_shape=jax.ShapeDtypeStruct((n, HIDDEN), jnp.float32),
        compiler_params=pltpu.CompilerParams(
            dimension_semantics=("parallel",)
        ),
    )(x, pe, gamma.reshape(1, HIDDEN), beta.reshape(1, HIDDEN))


def kernel(inputs_embeds, position_ids, pos_table, ln_gamma, ln_beta):
    b, s, h = inputs_embeds.shape
    ids_flat = position_ids.reshape(-1).astype(jnp.int32)
    pe = _sc_gather(pos_table, ids_flat)
    out = _tc_add_ln(inputs_embeds.reshape(-1, h), pe, ln_gamma, ln_beta)
    return out.reshape(b, s, h)


# speedup vs baseline: 1.0258x; 1.0258x over previous
import functools

import jax
import jax.numpy as jnp
from jax import lax
from jax.experimental import pallas as pl
from jax.experimental.pallas import tpu as pltpu
from jax.experimental.pallas import tpu_sc as plsc

MAX_POS = 4096
HIDDEN = 1024
EPS = 1e-12

NC = 2
NS = 16
NW = NC * NS

CHUNK = 32
NBUF = 3


def _sc_gather(pos_table, ids_flat):
    n_tokens = ids_flat.shape[0]
    b_per_w = n_tokens // NW
    n_ch = b_per_w // CHUNK
    mesh = plsc.VectorSubcoreMesh(core_axis_name="c", subcore_axis_name="s")

    @functools.partial(
        pl.kernel,
        mesh=mesh,
        out_type=jax.ShapeDtypeStruct((n_tokens, HIDDEN), jnp.float32),
        scratch_types=(
            [pltpu.VMEM((b_per_w,), jnp.int32)]
            + [pltpu.VMEM((CHUNK, HIDDEN), jnp.float32)] * NBUF
            + [pltpu.SemaphoreType.DMA] * (2 * NBUF)
        ),
    )
    def k(table_hbm, idx_hbm, out_hbm, idx_v, *scratch):
        bufs = list(scratch[:NBUF])
        gsem = list(scratch[NBUF : 2 * NBUF])
        ssem = list(scratch[2 * NBUF : 3 * NBUF])
        wid = lax.axis_index("s") * NC + lax.axis_index("c")
        base = wid * b_per_w
        pltpu.sync_copy(idx_hbm.at[pl.ds(base, b_per_w)], idx_v)

        gathers = [None] * n_ch
        stores = [None] * n_ch
        for c in range(n_ch):
            p = c % NBUF
            if c >= NBUF:
                stores[c - NBUF].wait()
            gathers[c] = pltpu.async_copy(
                table_hbm.at[idx_v.at[pl.ds(c * CHUNK, CHUNK)]],
                bufs[p],
                gsem[p],
            )
            if c >= 1:
                q = (c - 1) % NBUF
                gathers[c - 1].wait()
                stores[c - 1] = pltpu.async_copy(
                    bufs[q],
                    out_hbm.at[pl.ds(base + (c - 1) * CHUNK, CHUNK)],
                    ssem[q],
                )
        gathers[n_ch - 1].wait()
        stores[n_ch - 1] = pltpu.async_copy(
            bufs[(n_ch - 1) % NBUF],
            out_hbm.at[pl.ds(base + (n_ch - 1) * CHUNK, CHUNK)],
            ssem[(n_ch - 1) % NBUF],
        )
        for c in range(max(0, n_ch - NBUF), n_ch):
            stores[c].wait()

    return k(pos_table, ids_flat)


def _tc_add_ln(x, pe, gamma, beta):
    n = x.shape[0]
    bt = 1024
    grid = (n // bt,)

    def body(x_ref, p_ref, g_ref, b_ref, o_ref):
        e = x_ref[...] + p_ref[...]
        m = jnp.mean(e, axis=1, keepdims=True)
        d = e - m
        v = jnp.mean(d * d, axis=1, keepdims=True)
        o_ref[...] = d * lax.rsqrt(v + EPS) * g_ref[...] + b_ref[...]

    return pl.pallas_call(
        body,
        grid=grid,
        in_specs=[
            pl.BlockSpec((bt, HIDDEN), lambda i: (i, 0)),
            pl.BlockSpec((bt, HIDDEN), lambda i: (i, 0)),
            pl.BlockSpec((1, HIDDEN), lambda i: (0, 0)),
            pl.BlockSpec((1, HIDDEN), lambda i: (0, 0)),
        ],
        out_specs=pl.BlockSpec((bt, HIDDEN), lambda i: (i, 0)),
        out_shape=jax.ShapeDtypeStruct((n, HIDDEN), jnp.float32),
        compiler_params=pltpu.CompilerParams(
            dimension_semantics=("parallel",)
        ),
    )(x, pe, gamma.reshape(1, HIDDEN), beta.reshape(1, HIDDEN))


def kernel(inputs_embeds, position_ids, pos_table, ln_gamma, ln_beta):
    b, s, h = inputs_embeds.shape
    ids_flat = position_ids.reshape(-1).astype(jnp.int32)
    pe = _sc_gather(pos_table, ids_flat)
    out = _tc_add_ln(inputs_embeds.reshape(-1, h), pe, ln_gamma, ln_beta)
    return out.reshape(b, s, h)
